# baseline (speedup 1.0000x reference)
import functools

import jax
import jax.numpy as jnp
from jax import lax
from jax.experimental import pallas as pl
from jax.experimental.pallas import tpu as pltpu
from jax.experimental.pallas import tpu_sc as plsc

N = 10000
E = 320000
D = 128
PH = 512
L = 4

NC = 2
NS = 16
NW = NC * NS
LANES = 16
CH = 80
NCHUNK = E // CH
ROWS_PER_TILE = 624
ROWS_SUB = 48
ROWS_TAIL = N - NS * ROWS_PER_TILE


_sc_mesh = plsc.VectorSubcoreMesh(
    core_axis_name="c", subcore_axis_name="s", num_cores=NC, num_subcores=NS)

NBUF = 2
NIDX = 4


@functools.partial(
    pl.kernel,
    out_type=jax.ShapeDtypeStruct((NC, N, D), jnp.float32),
    mesh=_sc_mesh,
    scratch_types=(
        [pltpu.VMEM((CH,), jnp.int32) for _ in range(NIDX)]
        + [pltpu.VMEM((CH,), jnp.int32) for _ in range(NIDX)]
        + [pltpu.VMEM((CH, D), jnp.float32) for _ in range(NBUF)]
        + [pltpu.VMEM((CH, D), jnp.float32) for _ in range(NBUF)]
        + [pltpu.VMEM_SHARED((N, D), jnp.float32)]
        + [pltpu.SemaphoreType.DMA for _ in range(NIDX + 3 * NBUF)]
    ),
)
def _sc_edge_aggr(x_hbm, ea_hbm, src_hbm, dst_hbm, out_hbm, *scr):
    IS = scr[0:NIDX]
    ID = scr[NIDX:2 * NIDX]
    XR = scr[2 * NIDX:2 * NIDX + NBUF]
    ER = scr[2 * NIDX + NBUF:2 * NIDX + 2 * NBUF]
    aggr_sh = scr[2 * NIDX + 2 * NBUF]
    sems = scr[2 * NIDX + 2 * NBUF + 1:]
    SI = sems[0:NIDX]
    SG = sems[NIDX:NIDX + NBUF]
    SE = sems[NIDX + NBUF:NIDX + 2 * NBUF]
    SS = sems[NIDX + 2 * NBUF:NIDX + 3 * NBUF]

    c = lax.axis_index("c")
    s = lax.axis_index("s")
    wid = c * NS + s
    n = (NCHUNK + NW - 1 - wid) // NW

    def base_of(k):
        return (wid + k * NW) * CH

    def issue_idx(k, ib):
        base = base_of(k)
        pltpu.async_copy(src_hbm.at[pl.ds(base, CH)], IS[ib], SI[ib])
        pltpu.async_copy(dst_hbm.at[pl.ds(base, CH)], ID[ib], SI[ib])

    def wait_idx(ib):
        pltpu.make_async_copy(src_hbm.at[pl.ds(0, CH)], IS[ib], SI[ib]).wait()
        pltpu.make_async_copy(src_hbm.at[pl.ds(0, CH)], ID[ib], SI[ib]).wait()

    def issue_data(k, ib, db):
        pltpu.async_copy(x_hbm.at[IS[ib]], XR[db], SG[db])
        pltpu.async_copy(ea_hbm.at[pl.ds(base_of(k), CH)], ER[db], SE[db])

    def wait_data(db):
        pltpu.make_async_copy(ea_hbm.at[pl.ds(0, CH)], XR[db], SG[db]).wait()
        pltpu.make_async_copy(ea_hbm.at[pl.ds(0, CH)], ER[db], SE[db]).wait()

    def wait_scatter(db):
        pltpu.make_async_copy(ea_hbm.at[pl.ds(0, CH)], ER[db], SS[db]).wait()

    for j in range(2):
        @pl.when(j < n)
        def _pf(j=j):
            issue_idx(j, j)

    @pl.when(0 < n)
    def _b0():
        wait_idx(0)
        issue_data(0, 0, 0)

    zvec = jnp.zeros((LANES,), jnp.float32)

    def zrow(r, carry):
        for kk in range(D // LANES):
            ER[1][r, pl.ds(kk * LANES, LANES)] = zvec
        return carry

    lax.fori_loop(0, ROWS_SUB, zrow, 0)
    row0 = pl.multiple_of(s * ROWS_PER_TILE, 8)
    for j in range(ROWS_PER_TILE // ROWS_SUB):
        pltpu.sync_copy(ER[1].at[pl.ds(0, ROWS_SUB)],
                        aggr_sh.at[pl.ds(row0 + j * ROWS_SUB, ROWS_SUB)])

    @pl.when(s == NS - 1)
    def _zero_tail():
        pltpu.sync_copy(ER[1].at[pl.ds(0, ROWS_TAIL)],
                        aggr_sh.at[pl.ds(NS * ROWS_PER_TILE, ROWS_TAIL)])

    plsc.subcore_barrier()

    nsuper = (n + NIDX - 1) // NIDX

    def sstep(g, carry):
        for b in range(NIDX):
            k = g * NIDX + b
            db = b % NBUF
            db1 = (b + 1) % NBUF
            ib1 = (b + 1) % NIDX
            ib2 = (b + 2) % NIDX

            @pl.when(k < n)
            def _step(k=k, b=b, db=db, db1=db1, ib1=ib1, ib2=ib2):
                @pl.when(k + 2 < n)
                def _a():
                    issue_idx(k + 2, ib2)

                @pl.when(k >= 1)
                def _w1():
                    wait_scatter(db1)

                @pl.when(k + 1 < n)
                def _b():
                    wait_idx(ib1)
                    issue_data(k + 1, ib1, db1)

                wait_data(db)

                def crow(r, inner):
                    for kk in range(D // LANES):
                        sl = pl.ds(kk * LANES, LANES)
                        ER[db][r, sl] = jnp.maximum(
                            ER[db][r, sl] + XR[db][r, sl], 0.0)
                    return inner

                lax.fori_loop(0, CH, crow, 0)
                pltpu.async_copy(ER[db], aggr_sh.at[ID[b]], SS[db], add=True)

        return carry

    lax.fori_loop(0, nsuper, sstep, 0)

    for b in range(NBUF):
        @pl.when(((n - 1) % NBUF) == b)
        def _drain(b=b):
            wait_scatter(b)

    plsc.subcore_barrier()
    pltpu.sync_copy(aggr_sh.at[pl.ds(row0, ROWS_PER_TILE)],
                    out_hbm.at[c, pl.ds(row0, ROWS_PER_TILE)])

    @pl.when(s == NS - 1)
    def _write_tail():
        pltpu.sync_copy(aggr_sh.at[pl.ds(NS * ROWS_PER_TILE, ROWS_TAIL)],
                        out_hbm.at[c, pl.ds(NS * ROWS_PER_TILE, ROWS_TAIL)])


_BE = 2000


def _edge_mm_body(ew_ref, wcat_ref, becat_ref, o0, o1, o2, o3):
    acc = jnp.dot(ew_ref[...].astype(jnp.bfloat16),
                  wcat_ref[...].astype(jnp.bfloat16),
                  preferred_element_type=jnp.float32)
    outs = (o0, o1, o2, o3)
    for i in range(L):
        outs[i][...] = acc[:, i * D:(i + 1) * D] + becat_ref[:, i * D:(i + 1) * D]


def _edge_transforms(ew, wcat, becat):
    grid = (E // _BE,)
    return pl.pallas_call(
        _edge_mm_body,
        grid=grid,
        in_specs=[
            pl.BlockSpec((_BE, D), lambda i: (i, 0)),
            pl.BlockSpec((D, L * D), lambda i: (0, 0)),
            pl.BlockSpec((1, L * D), lambda i: (0, 0)),
        ],
        out_specs=[pl.BlockSpec((_BE, D), lambda i: (i, 0))] * L,
        out_shape=[jax.ShapeDtypeStruct((E, D), jnp.float32)] * L,
    )(ew, wcat, becat)


_BN = 2000


def _node_body(xa_ref, pa_ref, xb_ref, pb_ref, wn_ref, bn_ref, g_ref, b_ref,
               oa_ref, ob_ref):
    def one(x_ref, p_ref, o_ref):
        t = x_ref[...] + p_ref[0] + p_ref[1]
        h = jnp.dot(t, wn_ref[...], preferred_element_type=jnp.float32) + bn_ref[...]
        mu = jnp.mean(h, axis=1, keepdims=True)
        var = jnp.mean((h - mu) ** 2, axis=1, keepdims=True)
        hn = (h - mu) * lax.rsqrt(var + 1e-5) * g_ref[...] + b_ref[...]
        o_ref[...] = jnp.maximum(hn, 0.0)

    one(xa_ref, pa_ref, oa_ref)
    one(xb_ref, pb_ref, ob_ref)


def _node_update(xa, pa, xb, pb, wn, bn, g, b):
    grid = (N // _BN,)
    return pl.pallas_call(
        _node_body,
        grid=grid,
        in_specs=[
            pl.BlockSpec((_BN, D), lambda i: (i, 0)),
            pl.BlockSpec((NC, _BN, D), lambda i: (0, i, 0)),
            pl.BlockSpec((_BN, D), lambda i: (i, 0)),
            pl.BlockSpec((NC, _BN, D), lambda i: (0, i, 0)),
            pl.BlockSpec((D, D), lambda i: (0, 0)),
            pl.BlockSpec((1, D), lambda i: (0, 0)),
            pl.BlockSpec((1, D), lambda i: (0, 0)),
            pl.BlockSpec((1, D), lambda i: (0, 0)),
        ],
        out_specs=[pl.BlockSpec((_BN, D), lambda i: (i, 0))] * 2,
        out_shape=[jax.ShapeDtypeStruct((N, D), jnp.float32)] * 2,
    )(xa, pa, xb, pb, wn, bn, g, b)


def _pred_body(ha_ref, hb_ref, w1_ref, b1_ref, g_ref, b_ref, w2_ref, b2_ref,
               oa_ref, ob_ref):
    def one(h_ref, o_ref):
        h1 = jnp.dot(h_ref[...], w1_ref[...], preferred_element_type=jnp.float32) + b1_ref[...]
        mu = jnp.mean(h1, axis=1, keepdims=True)
        var = jnp.mean((h1 - mu) ** 2, axis=1, keepdims=True)
        hn = (h1 - mu) * lax.rsqrt(var + 1e-5) * g_ref[...] + b_ref[...]
        hn = jnp.maximum(hn, 0.0)
        o_ref[...] = jnp.dot(hn, w2_ref[...], preferred_element_type=jnp.float32) + b2_ref[...]

    one(ha_ref, oa_ref)
    one(hb_ref, ob_ref)


def _predictor(ha, hb, w1, b1, g, b, w2, b2):
    grid = (N // _BN,)
    return pl.pallas_call(
        _pred_body,
        grid=grid,
        in_specs=[
            pl.BlockSpec((_BN, D), lambda i: (i, 0)),
            pl.BlockSpec((_BN, D), lambda i: (i, 0)),
            pl.BlockSpec((D, PH), lambda i: (0, 0)),
            pl.BlockSpec((1, PH), lambda i: (0, 0)),
            pl.BlockSpec((1, PH), lambda i: (0, 0)),
            pl.BlockSpec((1, PH), lambda i: (0, 0)),
            pl.BlockSpec((PH, D), lambda i: (0, 0)),
            pl.BlockSpec((1, D), lambda i: (0, 0)),
        ],
        out_specs=[pl.BlockSpec((_BN, D), lambda i: (i, 0))] * 2,
        out_shape=[jax.ShapeDtypeStruct((N, D), jnp.float32)] * 2,
    )(ha, hb, w1, b1, g, b, w2, b2)


def kernel(x1, x2, edge_index_v1, edge_index_v2, edge_weight_v1, edge_weight_v2,
           Wn, bn, We, be, ln_g, ln_b, Wp1, bp1, gp, bp, Wp2, bp2):
    src1 = edge_index_v1[0].astype(jnp.int32)
    dst1 = edge_index_v1[1].astype(jnp.int32)
    src2 = edge_index_v2[0].astype(jnp.int32)
    dst2 = edge_index_v2[1].astype(jnp.int32)

    wcat = jnp.concatenate([We[i] for i in range(L)], axis=1)
    becat = be.reshape(1, L * D)

    ea1 = _edge_transforms(edge_weight_v1, wcat, becat)
    ea2 = _edge_transforms(edge_weight_v2, wcat, becat)

    h1, h2 = x1, x2
    for i in range(L):
        bn_i = bn[i].reshape(1, D)
        g_i = ln_g[i].reshape(1, D)
        b_i = ln_b[i].reshape(1, D)
        pa = _sc_edge_aggr(h1, ea1[i], src1, dst1)
        pb = _sc_edge_aggr(h2, ea2[i], src2, dst2)
        h1, h2 = _node_update(h1, pa, h2, pb, Wn[i], bn_i, g_i, b_i)

    bp1r = bp1.reshape(1, PH)
    gpr = gp.reshape(1, PH)
    bpr = bp.reshape(1, PH)
    bp2r = bp2.reshape(1, D)
    p1, p2 = _predictor(h1, h2, Wp1, bp1r, gpr, bpr, Wp2, bp2r)
    return (p1, p2, h1, h2)


# speedup vs baseline: 1.0217x; 1.0217x over previous
import functools

import jax
import jax.numpy as jnp
from jax import lax
from jax.experimental import pallas as pl
from jax.experimental.pallas import tpu as pltpu
from jax.experimental.pallas import tpu_sc as plsc

N = 10000
E = 320000
D = 128
PH = 512
L = 4

NC = 2
NS = 16
NW = NC * NS
LANES = 16
CH = 80
NCHUNK = E // CH
ROWS_PER_TILE = 624
ROWS_SUB = 48
ROWS_TAIL = N - NS * ROWS_PER_TILE


_sc_mesh = plsc.VectorSubcoreMesh(
    core_axis_name="c", subcore_axis_name="s", num_cores=NC, num_subcores=NS)

NBUF = 2
NIDX = 4


@functools.partial(
    pl.kernel,
    out_type=jax.ShapeDtypeStruct((NC, N, D), jnp.float32),
    mesh=_sc_mesh,
    scratch_types=(
        [pltpu.VMEM((CH,), jnp.int32) for _ in range(NIDX)]
        + [pltpu.VMEM((CH,), jnp.int32) for _ in range(NIDX)]
        + [pltpu.VMEM((CH, D), jnp.float32) for _ in range(NBUF)]
        + [pltpu.VMEM((CH, D), jnp.float32) for _ in range(NBUF)]
        + [pltpu.VMEM_SHARED((N, D), jnp.float32)]
        + [pltpu.SemaphoreType.DMA for _ in range(NIDX + 3 * NBUF)]
    ),
)
def _sc_edge_aggr(x_hbm, ea_hbm, src_hbm, dst_hbm, out_hbm, *scr):
    IS = scr[0:NIDX]
    ID = scr[NIDX:2 * NIDX]
    XR = scr[2 * NIDX:2 * NIDX + NBUF]
    ER = scr[2 * NIDX + NBUF:2 * NIDX + 2 * NBUF]
    aggr_sh = scr[2 * NIDX + 2 * NBUF]
    sems = scr[2 * NIDX + 2 * NBUF + 1:]
    SI = sems[0:NIDX]
    SG = sems[NIDX:NIDX + NBUF]
    SE = sems[NIDX + NBUF:NIDX + 2 * NBUF]
    SS = sems[NIDX + 2 * NBUF:NIDX + 3 * NBUF]

    c = lax.axis_index("c")
    s = lax.axis_index("s")
    wid = c * NS + s
    n = (NCHUNK + NW - 1 - wid) // NW

    def base_of(k):
        return (wid + k * NW) * CH

    def issue_idx(k, ib):
        base = base_of(k)
        pltpu.async_copy(src_hbm.at[pl.ds(base, CH)], IS[ib], SI[ib])
        pltpu.async_copy(dst_hbm.at[pl.ds(base, CH)], ID[ib], SI[ib])

    def wait_idx(ib):
        pltpu.make_async_copy(src_hbm.at[pl.ds(0, CH)], IS[ib], SI[ib]).wait()
        pltpu.make_async_copy(src_hbm.at[pl.ds(0, CH)], ID[ib], SI[ib]).wait()

    def issue_data(k, ib, db):
        pltpu.async_copy(x_hbm.at[IS[ib]], XR[db], SG[db])
        pltpu.async_copy(ea_hbm.at[pl.ds(base_of(k), CH)], ER[db], SE[db])

    def wait_data(db):
        pltpu.make_async_copy(ea_hbm.at[pl.ds(0, CH)], XR[db], SG[db]).wait()
        pltpu.make_async_copy(ea_hbm.at[pl.ds(0, CH)], ER[db], SE[db]).wait()

    def wait_scatter(db):
        pltpu.make_async_copy(ea_hbm.at[pl.ds(0, CH)], ER[db], SS[db]).wait()

    for j in range(2):
        @pl.when(j < n)
        def _pf(j=j):
            issue_idx(j, j)

    @pl.when(0 < n)
    def _b0():
        wait_idx(0)
        issue_data(0, 0, 0)

    zvec = jnp.zeros((LANES,), jnp.float32)

    def zrow(r, carry):
        for kk in range(D // LANES):
            ER[1][r, pl.ds(kk * LANES, LANES)] = zvec
        return carry

    lax.fori_loop(0, ROWS_SUB, zrow, 0)
    row0 = pl.multiple_of(s * ROWS_PER_TILE, 8)
    for j in range(ROWS_PER_TILE // ROWS_SUB):
        pltpu.sync_copy(ER[1].at[pl.ds(0, ROWS_SUB)],
                        aggr_sh.at[pl.ds(row0 + j * ROWS_SUB, ROWS_SUB)])

    @pl.when(s == NS - 1)
    def _zero_tail():
        pltpu.sync_copy(ER[1].at[pl.ds(0, ROWS_TAIL)],
                        aggr_sh.at[pl.ds(NS * ROWS_PER_TILE, ROWS_TAIL)])

    plsc.subcore_barrier()

    nsuper = (n + NIDX - 1) // NIDX

    def sstep(g, carry):
        for b in range(NIDX):
            k = g * NIDX + b
            db = b % NBUF
            db1 = (b + 1) % NBUF
            ib1 = (b + 1) % NIDX
            ib2 = (b + 2) % NIDX

            @pl.when(k < n)
            def _step(k=k, b=b, db=db, db1=db1, ib1=ib1, ib2=ib2):
                @pl.when(k + 2 < n)
                def _a():
                    issue_idx(k + 2, ib2)

                @pl.when(k >= 1)
                def _w1():
                    wait_scatter(db1)

                @pl.when(k + 1 < n)
                def _b():
                    wait_idx(ib1)
                    issue_data(k + 1, ib1, db1)

                wait_data(db)

                def crow(r, inner):
                    for kk in range(D // LANES):
                        sl = pl.ds(kk * LANES, LANES)
                        ER[db][r, sl] = jnp.maximum(
                            ER[db][r, sl] + XR[db][r, sl], 0.0)
                    return inner

                lax.fori_loop(0, CH, crow, 0)
                pltpu.async_copy(ER[db], aggr_sh.at[ID[b]], SS[db], add=True)

        return carry

    lax.fori_loop(0, nsuper, sstep, 0)

    for b in range(NBUF):
        @pl.when(((n - 1) % NBUF) == b)
        def _drain(b=b):
            wait_scatter(b)

    plsc.subcore_barrier()
    pltpu.sync_copy(aggr_sh.at[pl.ds(row0, ROWS_PER_TILE)],
                    out_hbm.at[c, pl.ds(row0, ROWS_PER_TILE)])

    @pl.when(s == NS - 1)
    def _write_tail():
        pltpu.sync_copy(aggr_sh.at[pl.ds(NS * ROWS_PER_TILE, ROWS_TAIL)],
                        out_hbm.at[c, pl.ds(NS * ROWS_PER_TILE, ROWS_TAIL)])


_BE = 2000


def _edge_mm_body(ew_ref, wcat_ref, becat_ref, o0, o1, o2, o3):
    acc = jnp.dot(ew_ref[...].astype(jnp.bfloat16),
                  wcat_ref[...].astype(jnp.bfloat16),
                  preferred_element_type=jnp.float32)
    outs = (o0, o1, o2, o3)
    for i in range(L):
        outs[i][...] = acc[:, i * D:(i + 1) * D] + becat_ref[:, i * D:(i + 1) * D]


def _edge_transforms(ew, wcat, becat):
    grid = (E // _BE,)
    return pl.pallas_call(
        _edge_mm_body,
        grid=grid,
        in_specs=[
            pl.BlockSpec((_BE, D), lambda i: (i, 0)),
            pl.BlockSpec((D, L * D), lambda i: (0, 0)),
            pl.BlockSpec((1, L * D), lambda i: (0, 0)),
        ],
        out_specs=[pl.BlockSpec((_BE, D), lambda i: (i, 0))] * L,
        out_shape=[jax.ShapeDtypeStruct((E, D), jnp.float32)] * L,
    )(ew, wcat, becat)


_BN = 2000


def _node_body(x_ref, p_ref, wn_ref, bn_ref, g_ref, b_ref, o_ref):
    t = x_ref[...] + p_ref[0] + p_ref[1]
    h = jnp.dot(t, wn_ref[...], preferred_element_type=jnp.float32) + bn_ref[...]
    mu = jnp.mean(h, axis=1, keepdims=True)
    var = jnp.mean((h - mu) ** 2, axis=1, keepdims=True)
    hn = (h - mu) * lax.rsqrt(var + 1e-5) * g_ref[...] + b_ref[...]
    o_ref[...] = jnp.maximum(hn, 0.0)


def _node_update(x, p, wn, bn, g, b):
    grid = (N // _BN,)
    return pl.pallas_call(
        _node_body,
        grid=grid,
        in_specs=[
            pl.BlockSpec((_BN, D), lambda i: (i, 0)),
            pl.BlockSpec((NC, _BN, D), lambda i: (0, i, 0)),
            pl.BlockSpec((D, D), lambda i: (0, 0)),
            pl.BlockSpec((1, D), lambda i: (0, 0)),
            pl.BlockSpec((1, D), lambda i: (0, 0)),
            pl.BlockSpec((1, D), lambda i: (0, 0)),
        ],
        out_specs=pl.BlockSpec((_BN, D), lambda i: (i, 0)),
        out_shape=jax.ShapeDtypeStruct((N, D), jnp.float32),
    )(x, p, wn, bn, g, b)


def _pred_body(h_ref, w1_ref, b1_ref, g_ref, b_ref, w2_ref, b2_ref, o_ref):
    h1 = jnp.dot(h_ref[...], w1_ref[...], preferred_element_type=jnp.float32) + b1_ref[...]
    mu = jnp.mean(h1, axis=1, keepdims=True)
    var = jnp.mean((h1 - mu) ** 2, axis=1, keepdims=True)
    hn = (h1 - mu) * lax.rsqrt(var + 1e-5) * g_ref[...] + b_ref[...]
    hn = jnp.maximum(hn, 0.0)
    o_ref[...] = jnp.dot(hn, w2_ref[...], preferred_element_type=jnp.float32) + b2_ref[...]


def _predictor(h, w1, b1, g, b, w2, b2):
    grid = (N // _BN,)
    return pl.pallas_call(
        _pred_body,
        grid=grid,
        in_specs=[
            pl.BlockSpec((_BN, D), lambda i: (i, 0)),
            pl.BlockSpec((D, PH), lambda i: (0, 0)),
            pl.BlockSpec((1, PH), lambda i: (0, 0)),
            pl.BlockSpec((1, PH), lambda i: (0, 0)),
            pl.BlockSpec((1, PH), lambda i: (0, 0)),
            pl.BlockSpec((PH, D), lambda i: (0, 0)),
            pl.BlockSpec((1, D), lambda i: (0, 0)),
        ],
        out_specs=pl.BlockSpec((_BN, D), lambda i: (i, 0)),
        out_shape=jax.ShapeDtypeStruct((N, D), jnp.float32),
    )(h, w1, b1, g, b, w2, b2)


def kernel(x1, x2, edge_index_v1, edge_index_v2, edge_weight_v1, edge_weight_v2,
           Wn, bn, We, be, ln_g, ln_b, Wp1, bp1, gp, bp, Wp2, bp2):
    src1 = edge_index_v1[0].astype(jnp.int32)
    dst1 = edge_index_v1[1].astype(jnp.int32)
    src2 = edge_index_v2[0].astype(jnp.int32)
    dst2 = edge_index_v2[1].astype(jnp.int32)

    wcat = jnp.concatenate([We[i] for i in range(L)], axis=1)
    becat = be.reshape(1, L * D)

    ea1 = _edge_transforms(edge_weight_v1, wcat, becat)
    ea2 = _edge_transforms(edge_weight_v2, wcat, becat)

    h1, h2 = x1, x2
    for i in range(L):
        bn_i = bn[i].reshape(1, D)
        g_i = ln_g[i].reshape(1, D)
        b_i = ln_b[i].reshape(1, D)
        pa = _sc_edge_aggr(h1, ea1[i], src1, dst1)
        pb = _sc_edge_aggr(h2, ea2[i], src2, dst2)
        h1 = _node_update(h1, pa, Wn[i], bn_i, g_i, b_i)
        h2 = _node_update(h2, pb, Wn[i], bn_i, g_i, b_i)

    bp1r = bp1.reshape(1, PH)
    gpr = gp.reshape(1, PH)
    bpr = bp.reshape(1, PH)
    bp2r = bp2.reshape(1, D)
    p1 = _predictor(h1, Wp1, bp1r, gpr, bpr, Wp2, bp2r)
    p2 = _predictor(h2, Wp1, bp1r, gpr, bpr, Wp2, bp2r)
    return (p1, p2, h1, h2)
